# baseline (speedup 1.0000x reference)
import functools

import jax
import jax.numpy as jnp
from jax import lax
from jax.experimental import pallas as pl
from jax.experimental.pallas import tpu as pltpu
from jax.experimental.pallas import tpu_sc as plsc

BATCH = 4096
EMBED = 64
HIDDEN = 64


def _make_sc_gather(V2, B):
    info = plsc.get_sparse_core_info()
    NC, NS, L = info.num_cores, info.num_subcores, info.num_lanes
    NW = NC * NS
    assert B % (8 * NW) == 0
    b_per_w = B // NW
    mesh = plsc.VectorSubcoreMesh(core_axis_name="c", subcore_axis_name="s")

    @functools.partial(
        pl.kernel,
        mesh=mesh,
        out_type=jax.ShapeDtypeStruct((B, 128), jnp.float32),
        scratch_types=[
            pltpu.VMEM((b_per_w,), jnp.int32),
            pltpu.VMEM((b_per_w,), jnp.int32),
            pltpu.VMEM((b_per_w, 128), jnp.float32),
            pltpu.SemaphoreType.DMA,
        ],
    )
    def gather(table_hbm, idx_hbm, out_hbm, idx_v, idx2_v, rows_v, sem):
        wid = lax.axis_index("s") * NC + lax.axis_index("c")
        base = wid * b_per_w
        pltpu.sync_copy(idx_hbm.at[pl.ds(base, b_per_w)], idx_v)
        for k in range(b_per_w // L):
            sl = pl.ds(k * L, L)
            idx2_v[sl] = lax.shift_right_logical(idx_v[sl], 1)
        pltpu.async_copy(table_hbm.at[idx2_v], rows_v, sem).wait()
        pltpu.sync_copy(rows_v, out_hbm.at[pl.ds(base, b_per_w)])

    return gather


def _gru_body(x2_ref, idx_ref, h_ref, wr_ref, wz_ref, wn_ref, ur_ref,
              uz_ref, un_ref, br_ref, bz_ref, bin_ref, bhn_ref, out_ref):
    x2 = x2_ref[...]
    par = (idx_ref[...] & 1) == 1
    x = jnp.where(par, x2[:, 64:], x2[:, :64])
    h = h_ref[...]
    f32 = jnp.float32
    r = jax.nn.sigmoid(
        jnp.dot(x, wr_ref[...], preferred_element_type=f32)
        + jnp.dot(h, ur_ref[...], preferred_element_type=f32)
        + br_ref[...]
    )
    z = jax.nn.sigmoid(
        jnp.dot(x, wz_ref[...], preferred_element_type=f32)
        + jnp.dot(h, uz_ref[...], preferred_element_type=f32)
        + bz_ref[...]
    )
    hn = jnp.dot(h, un_ref[...], preferred_element_type=f32) + bhn_ref[...]
    n = jnp.tanh(
        jnp.dot(x, wn_ref[...], preferred_element_type=f32)
        + bin_ref[...]
        + r * hn
    )
    out_ref[...] = (1.0 - z) * n + z * h


def kernel(input_data, batch_size, hidden, embedding_matrix, W_ih, W_hh,
           b_ih, b_hh):
    V, D = embedding_matrix.shape
    H = HIDDEN
    idx = input_data.astype(jnp.int32)
    table2 = embedding_matrix.reshape(V // 2, 2 * D)

    gather = _make_sc_gather(V // 2, BATCH)
    x2 = gather(table2, idx)

    wi = W_ih.T
    wh = W_hh.T
    wr, wz, wn = wi[:, :H], wi[:, H:2 * H], wi[:, 2 * H:]
    ur, uz, un = wh[:, :H], wh[:, H:2 * H], wh[:, 2 * H:]
    br = (b_ih[:H] + b_hh[:H]).reshape(1, H)
    bz = (b_ih[H:2 * H] + b_hh[H:2 * H]).reshape(1, H)
    bin_ = b_ih[2 * H:].reshape(1, H)
    bhn = b_hh[2 * H:].reshape(1, H)

    h0 = hidden[0]
    h1 = pl.pallas_call(
        _gru_body,
        out_shape=jax.ShapeDtypeStruct((BATCH, HIDDEN), jnp.float32),
    )(x2, idx.reshape(BATCH, 1), h0, wr, wz, wn, ur, uz, un, br, bz, bin_,
      bhn)
    out = h1[None, :, :]
    return (out, out)


# speedup vs baseline: 2.2754x; 2.2754x over previous
import functools

import jax
import jax.numpy as jnp
from jax import lax
from jax.experimental import pallas as pl
from jax.experimental.pallas import tpu as pltpu
from jax.experimental.pallas import tpu_sc as plsc

BATCH = 4096
EMBED = 64
HIDDEN = 64


def _make_sc_gather(V, D, B):
    info = plsc.get_sparse_core_info()
    NC, NS = info.num_cores, info.num_subcores
    NW = NC * NS
    assert B % (8 * NW) == 0
    b_per_w = B // NW
    mesh = plsc.VectorSubcoreMesh(core_axis_name="c", subcore_axis_name="s")

    @functools.partial(
        pl.kernel,
        mesh=mesh,
        out_type=jax.ShapeDtypeStruct((B, D), jnp.float32),
        scratch_types=[
            pltpu.VMEM((b_per_w,), jnp.int32),
            pltpu.VMEM((b_per_w, D), jnp.float32),
            pltpu.SemaphoreType.DMA,
        ],
        compiler_params=pltpu.CompilerParams(needs_layout_passes=False),
    )
    def gather(table_hbm, idx_hbm, out_hbm, idx_v, rows_v, sem):
        wid = lax.axis_index("s") * NC + lax.axis_index("c")
        base = wid * b_per_w
        pltpu.sync_copy(idx_hbm.at[pl.ds(base, b_per_w)], idx_v)
        L = 16
        lane = lax.iota(jnp.int32, L)
        for g in range(b_per_w // L):
            vec = idx_v[pl.ds(g * L, L)]
            for l in range(L):
                i = jnp.sum(jnp.where(lane == l, vec, 0))
                pltpu.make_async_copy(
                    table_hbm.at[pl.ds(i, 1)],
                    rows_v.at[pl.ds(g * L + l, 1)],
                    sem,
                ).start()
        pltpu.make_async_copy(
            table_hbm.at[pl.ds(0, b_per_w)], rows_v, sem
        ).wait()
        pltpu.sync_copy(rows_v, out_hbm.at[pl.ds(base, b_per_w)])

    return gather


def _gru_body(x_ref, h_ref, wr_ref, wz_ref, wn_ref, ur_ref, uz_ref, un_ref,
              br_ref, bz_ref, bin_ref, bhn_ref, out_ref):
    x = x_ref[...]
    h = h_ref[...]
    f32 = jnp.float32
    r = jax.nn.sigmoid(
        jnp.dot(x, wr_ref[...], preferred_element_type=f32)
        + jnp.dot(h, ur_ref[...], preferred_element_type=f32)
        + br_ref[...]
    )
    z = jax.nn.sigmoid(
        jnp.dot(x, wz_ref[...], preferred_element_type=f32)
        + jnp.dot(h, uz_ref[...], preferred_element_type=f32)
        + bz_ref[...]
    )
    hn = jnp.dot(h, un_ref[...], preferred_element_type=f32) + bhn_ref[...]
    n = jnp.tanh(
        jnp.dot(x, wn_ref[...], preferred_element_type=f32)
        + bin_ref[...]
        + r * hn
    )
    out_ref[...] = (1.0 - z) * n + z * h


def kernel(input_data, batch_size, hidden, embedding_matrix, W_ih, W_hh,
           b_ih, b_hh):
    V, D = embedding_matrix.shape
    H = HIDDEN
    idx = input_data.astype(jnp.int32)

    gather = _make_sc_gather(V, D, BATCH)
    x = gather(embedding_matrix, idx)

    wi = W_ih.T
    wh = W_hh.T
    wr, wz, wn = wi[:, :H], wi[:, H:2 * H], wi[:, 2 * H:]
    ur, uz, un = wh[:, :H], wh[:, H:2 * H], wh[:, 2 * H:]
    br = (b_ih[:H] + b_hh[:H]).reshape(1, H)
    bz = (b_ih[H:2 * H] + b_hh[H:2 * H]).reshape(1, H)
    bin_ = b_ih[2 * H:].reshape(1, H)
    bhn = b_hh[2 * H:].reshape(1, H)

    h0 = hidden[0]
    h1 = pl.pallas_call(
        _gru_body,
        out_shape=jax.ShapeDtypeStruct((BATCH, HIDDEN), jnp.float32),
    )(x, h0, wr, wz, wn, ur, uz, un, br, bz, bin_, bhn)
    out = h1[None, :, :]
    return (out, out)
